# baseline (speedup 1.0000x reference)
import jax
import jax.numpy as jnp
from jax import lax
from jax.experimental import pallas as pl
from jax.experimental.pallas import tpu as pltpu
from jax.experimental.pallas import tpu_sc as plsc

EPS_BN = 1e-5
N = 10000
E = 160000
D = 256
HALF = 128
NH = 5000
CK = 64
NT = 16
NC = 2
EPAD = NT * 10240
ET = EPAD // NT
CAP = ET + 4 * CK + 16
APAD = 5120
ZR = APAD // NT
BM = 200
NBM = N // BM
BN = 400
NB = N // BN


def _tables_body(x_ref, wt_ref, t_ref, s_ref):
    xb = x_ref[...]
    for t in range(4):
        m = jnp.maximum(xb + wt_ref[t], 0.0)
        t_ref[0, t] = m[:, :HALF]
        t_ref[1, t] = m[:, HALF:]
    s_ref[...] = jnp.maximum(xb + wt_ref[4], 0.0)


def _build_tables(x, w_type):
    return pl.pallas_call(
        _tables_body,
        grid=(NB,),
        in_specs=[
            pl.BlockSpec((BN, D), lambda i: (i, 0)),
            pl.BlockSpec((8, D), lambda i: (0, 0)),
        ],
        out_specs=[
            pl.BlockSpec((NC, 4, BN, HALF), lambda i: (0, 0, i, 0)),
            pl.BlockSpec((BN, D), lambda i: (i, 0)),
        ],
        out_shape=[
            jax.ShapeDtypeStruct((NC, 4, N, HALF), jnp.float32),
            jax.ShapeDtypeStruct((N, D), jnp.float32),
        ],
    )(x, w_type)


def _sc_body(t_hbm, src_hbm, attr_hbm, dst_hbm, zeros_hbm, out_hbm,
             src_v, attr_v, dst_v, l0g, l0d, l1g, l1d, buf0, buf1, aggr_s,
             sem0, sem1):
    cid = lax.axis_index("c")
    sid = lax.axis_index("s")

    pltpu.sync_copy(src_hbm.at[sid], src_v)
    pltpu.sync_copy(attr_hbm.at[sid], attr_v)
    pltpu.sync_copy(dst_hbm.at[sid], dst_v)

    base = cid * (4 * N)

    lane = lax.iota(jnp.int32, 16)

    def part_body(i, carry):
        off0, off1 = carry
        c = i * 16
        s16 = src_v[pl.ds(c, 16)]
        a16 = attr_v[pl.ds(c, 16)]
        d16 = dst_v[pl.ds(c, 16)]
        g16 = a16 * N + s16 + base
        m0 = d16 < NH
        m1 = jnp.logical_not(m0)
        z16 = jnp.where(m0, 1, 0).astype(jnp.int32)
        inc0 = plsc.cumsum(z16)
        exc0 = inc0 - z16
        idx0 = off0 + exc0
        idx1 = off1 + (lane - exc0)
        plsc.store_scatter(l0g, [idx0], g16, mask=m0)
        plsc.store_scatter(l0d, [idx0], d16, mask=m0)
        plsc.store_scatter(l1g, [idx1], g16, mask=m1)
        plsc.store_scatter(l1d, [idx1], d16 - NH, mask=m1)
        n0 = jnp.max(inc0)
        return off0 + n0, off1 + (16 - n0)

    off0, off1 = lax.fori_loop(0, ET // 16, part_body,
                               (jnp.int32(0), jnp.int32(0)))

    def gather(lg, j, buf, sem):
        pltpu.async_copy(t_hbm.at[lg.at[pl.ds(CK * j, CK)]], buf, sem)

    def wait(buf, sem):
        pltpu.make_async_copy(t_hbm.at[l0g.at[pl.ds(0, CK)]], buf,
                              sem).wait()

    def scat(ld, j, buf):
        pltpu.sync_copy(buf, aggr_s.at[ld.at[pl.ds(CK * j, CK)]], add=True)

    for p, (lg, ld, off) in enumerate(((l0g, l0d, off0), (l1g, l1d, off1))):
        def pad_body(k, _):
            lg[pl.ds(off + k * 16, 16)] = jnp.zeros((16,), jnp.int32)
            ld[pl.ds(off + k * 16, 16)] = jnp.full((16,), NH, jnp.int32)
            return 0

        lax.fori_loop(0, (4 * CK) // 16, pad_body, 0)

        pltpu.sync_copy(zeros_hbm, aggr_s.at[pl.ds(sid * ZR, ZR)])
        plsc.subcore_barrier()

        tp = (off + (CK - 1)) // CK
        tpp = (tp + 1) // 2

        gather(lg, 0, buf0, sem0)
        gather(lg, 1, buf1, sem1)

        def pair_body(i, _):
            j = 2 * i
            wait(buf0, sem0)
            scat(ld, j, buf0)
            gather(lg, j + 2, buf0, sem0)
            wait(buf1, sem1)
            scat(ld, j + 1, buf1)
            gather(lg, j + 3, buf1, sem1)
            return 0

        lax.fori_loop(0, tpp, pair_body, 0)
        wait(buf0, sem0)
        wait(buf1, sem1)

        plsc.subcore_barrier()
        pltpu.sync_copy(aggr_s.at[pl.ds(sid * ZR, ZR)],
                        out_hbm.at[pl.ds((2 * p + cid) * APAD + sid * ZR,
                                         ZR)])


def _sc_aggregate(table, src3, attr3, dst3, zeros):
    mesh = plsc.VectorSubcoreMesh(core_axis_name="c", subcore_axis_name="s")
    call = pl.kernel(
        _sc_body,
        out_type=jax.ShapeDtypeStruct((2 * NC * APAD, HALF), jnp.float32),
        mesh=mesh,
        compiler_params=pltpu.CompilerParams(use_tc_tiling_on_sc=False,
                                             needs_layout_passes=False),
        scratch_types=[
            pltpu.VMEM((ET,), jnp.int32),
            pltpu.VMEM((ET,), jnp.int32),
            pltpu.VMEM((ET,), jnp.int32),
            pltpu.VMEM((CAP,), jnp.int32),
            pltpu.VMEM((CAP,), jnp.int32),
            pltpu.VMEM((CAP,), jnp.int32),
            pltpu.VMEM((CAP,), jnp.int32),
            pltpu.VMEM((CK, HALF), jnp.float32),
            pltpu.VMEM((CK, HALF), jnp.float32),
            pltpu.VMEM_SHARED((APAD, HALF), jnp.float32),
            pltpu.SemaphoreType.DMA,
            pltpu.SemaphoreType.DMA,
        ],
    )
    out = call(table.reshape(NC * 4 * N, HALF), src3, attr3, dst3, zeros)
    return out.reshape(2, NC, APAD, HALF)


def _prep_edges(edge_index, edge_attr):
    src = edge_index[0]
    dst = edge_index[1]
    a0 = edge_attr[:, 0]
    pad = EPAD - E
    src = jnp.concatenate([src, jnp.zeros((pad,), src.dtype)])
    a0 = jnp.concatenate([a0, jnp.zeros((pad,), a0.dtype)])
    dst = jnp.concatenate([dst, jnp.full((pad,), N, dst.dtype)])
    return (src.reshape(NT, ET), a0.reshape(NT, ET), dst.reshape(NT, ET))


def _mlp_body(agg_ref, s_ref, w1_ref, b1_ref, w2_ref, b2_ref, h_ref, st_ref):
    a = jnp.concatenate([agg_ref[0, 0], agg_ref[0, 1]], axis=1) + s_ref[...]
    z = lax.dot_general(a, w1_ref[...], (((1,), (1,)), ((), ())),
                        preferred_element_type=jnp.float32) + b1_ref[...]
    z = jnp.maximum(z, 0.0)
    h = lax.dot_general(z, w2_ref[...], (((1,), (1,)), ((), ())),
                        preferred_element_type=jnp.float32) + b2_ref[...]
    h_ref[...] = h
    su = jnp.sum(h, axis=0, keepdims=True)
    sq = jnp.sum(h * h, axis=0, keepdims=True)
    part = jnp.concatenate([su, sq, jnp.zeros((6, D), jnp.float32)], axis=0)

    @pl.when(pl.program_id(0) == 0)
    def _():
        st_ref[...] = part

    @pl.when(pl.program_id(0) > 0)
    def _():
        st_ref[...] = st_ref[...] + part


def _mlp(agg, s, w1, b1, w2, b2):
    nhb = NH // BM
    return pl.pallas_call(
        _mlp_body,
        grid=(NBM,),
        in_specs=[
            pl.BlockSpec((1, NC, BM, HALF),
                         lambda i: (i // nhb, 0, i % nhb, 0)),
            pl.BlockSpec((BM, D), lambda i: (i, 0)),
            pl.BlockSpec((2 * D, D), lambda i: (0, 0)),
            pl.BlockSpec((1, 2 * D), lambda i: (0, 0)),
            pl.BlockSpec((D, 2 * D), lambda i: (0, 0)),
            pl.BlockSpec((1, D), lambda i: (0, 0)),
        ],
        out_specs=[
            pl.BlockSpec((BM, D), lambda i: (i, 0)),
            pl.BlockSpec((8, D), lambda i: (0, 0)),
        ],
        out_shape=[
            jax.ShapeDtypeStruct((N, D), jnp.float32),
            jax.ShapeDtypeStruct((8, D), jnp.float32),
        ],
    )(agg, s, w1, b1, w2, b2)


def _norm_body(h_ref, st_ref, g_ref, bt_ref, o_ref):
    mean = st_ref[0:1, :] * (1.0 / N)
    msq = st_ref[1:2, :] * (1.0 / N)
    var = msq - mean * mean
    inv = lax.rsqrt(var + EPS_BN)
    o_ref[...] = jnp.maximum(
        (h_ref[...] - mean) * inv * g_ref[...] + bt_ref[...], 0.0)


def _norm(h, st, gamma, beta):
    return pl.pallas_call(
        _norm_body,
        grid=(NB,),
        in_specs=[
            pl.BlockSpec((BN, D), lambda i: (i, 0)),
            pl.BlockSpec((8, D), lambda i: (0, 0)),
            pl.BlockSpec((1, D), lambda i: (0, 0)),
            pl.BlockSpec((1, D), lambda i: (0, 0)),
        ],
        out_specs=pl.BlockSpec((BN, D), lambda i: (i, 0)),
        out_shape=jax.ShapeDtypeStruct((N, D), jnp.float32),
    )(h, st, gamma, beta)


def _graph(x, edge_index, edge_attr, w_type, w1, b1, w2, b2, gamma, beta,
           zeros):
    table, s = _build_tables(x, w_type)
    src3, attr3, dst3 = _prep_edges(edge_index, edge_attr)
    agg = _sc_aggregate(table, src3, attr3, dst3, zeros)
    h, st = _mlp(agg, s, w1, b1.reshape(1, 2 * D), w2, b2.reshape(1, D))
    return _norm(h, st, gamma.reshape(1, D), beta.reshape(1, D))


def kernel(xA, edge_indexA, edge_attrA, xB, edge_indexB, edge_attrB,
           W_type, W1, b1, W2, b2, gamma, beta):
    zeros = jnp.zeros((ZR, HALF), jnp.float32)
    outA = _graph(xA, edge_indexA, edge_attrA, W_type, W1, b1, W2, b2,
                  gamma, beta, zeros)
    outB = _graph(xB, edge_indexB, edge_attrB, W_type, W1, b1, W2, b2,
                  gamma, beta, zeros)
    return (outA, outB)


# speedup vs baseline: 1.9333x; 1.9333x over previous
import jax
import jax.numpy as jnp
from jax import lax
from jax.experimental import pallas as pl
from jax.experimental.pallas import tpu as pltpu
from jax.experimental.pallas import tpu_sc as plsc

EPS_BN = 1e-5
N = 10000
E = 160000
D = 256
QW = 64
NQ = 4
NT = 16
NC = 2
NP = 2
ET = 10240
EPAD = NT * ET
CK = 128
NCH = ET // CK
NPAD = 10112
ZR = NPAD // NT
BN = 400
NB = N // BN


def _tables_body(x_ref, wt_ref, t_ref, s_ref):
    xb = x_ref[...]
    for t in range(4):
        m = jnp.maximum(xb + wt_ref[t], 0.0)
        for q in range(NQ):
            t_ref[q, t] = m[:, q * QW:(q + 1) * QW]
    s_ref[...] = jnp.maximum(xb + wt_ref[4], 0.0)


def _build_tables(x, w_type):
    return pl.pallas_call(
        _tables_body,
        grid=(NB,),
        in_specs=[
            pl.BlockSpec((BN, D), lambda i: (i, 0)),
            pl.BlockSpec((8, D), lambda i: (0, 0)),
        ],
        out_specs=[
            pl.BlockSpec((NQ, 4, BN, QW), lambda i: (0, 0, i, 0)),
            pl.BlockSpec((BN, D), lambda i: (i, 0)),
        ],
        out_shape=[
            jax.ShapeDtypeStruct((NQ, 4, N, QW), jnp.float32),
            jax.ShapeDtypeStruct((N, D), jnp.float32),
        ],
    )(x, w_type)


def _sc_body(t_hbm, src_hbm, attr_hbm, dst_hbm, zeros_hbm, out_hbm,
             gidx_v, attr_v, dst_v, buf0, buf1, buf2, buf3, aggr_s,
             sem0, sem1, sem2, sem3):
    cid = lax.axis_index("c")
    sid = lax.axis_index("s")

    pltpu.sync_copy(src_hbm.at[sid], gidx_v)
    pltpu.sync_copy(attr_hbm.at[sid], attr_v)
    pltpu.sync_copy(dst_hbm.at[sid], dst_v)

    base = cid * (4 * N)

    def idx_body(i, _):
        c = i * 16
        s16 = gidx_v[pl.ds(c, 16)]
        a16 = attr_v[pl.ds(c, 16)]
        gidx_v[pl.ds(c, 16)] = a16 * N + s16 + base
        return 0

    def bump_body(i, _):
        c = i * 16
        gidx_v[pl.ds(c, 16)] = gidx_v[pl.ds(c, 16)] + 2 * (4 * N)
        return 0

    lax.fori_loop(0, ET // 16, idx_body, 0)

    def gather(j, buf, sem):
        pltpu.async_copy(t_hbm.at[gidx_v.at[pl.ds(CK * j, CK)]], buf, sem)

    def wait(buf, sem):
        pltpu.make_async_copy(t_hbm.at[gidx_v.at[pl.ds(0, CK)]], buf,
                              sem).wait()

    def scat(j, buf):
        pltpu.sync_copy(buf, aggr_s.at[dst_v.at[pl.ds(CK * j, CK)]],
                        add=True)

    for p in range(NP):
        if p > 0:
            lax.fori_loop(0, ET // 16, bump_body, 0)

        pltpu.sync_copy(zeros_hbm, aggr_s.at[pl.ds(sid * ZR, ZR)])
        plsc.subcore_barrier()

        bufs = (buf0, buf1, buf2, buf3)
        sems = (sem0, sem1, sem2, sem3)
        for k in range(4):
            gather(k, bufs[k], sems[k])

        def loop_body(i, _):
            j = 4 * i
            for k in range(4):
                wait(bufs[k], sems[k])
                scat(j + k, bufs[k])
                gather(j + k + 4, bufs[k], sems[k])
            return 0

        lax.fori_loop(0, NCH // 4 - 1, loop_body, 0)
        for k in range(4):
            wait(bufs[k], sems[k])
            scat(NCH - 4 + k, bufs[k])

        plsc.subcore_barrier()
        q = 2 * p + cid
        pltpu.sync_copy(aggr_s.at[pl.ds(sid * ZR, ZR)],
                        out_hbm.at[pl.ds(q * NPAD + sid * ZR, ZR)])


def _sc_aggregate(table, src3, attr3, dst3, zeros):
    mesh = plsc.VectorSubcoreMesh(core_axis_name="c", subcore_axis_name="s")
    call = pl.kernel(
        _sc_body,
        out_type=jax.ShapeDtypeStruct((NQ * NPAD, QW), jnp.float32),
        mesh=mesh,
        compiler_params=pltpu.CompilerParams(use_tc_tiling_on_sc=False),
        scratch_types=[
            pltpu.VMEM((ET,), jnp.int32),
            pltpu.VMEM((ET,), jnp.int32),
            pltpu.VMEM((ET,), jnp.int32),
            pltpu.VMEM((CK, QW), jnp.float32),
            pltpu.VMEM((CK, QW), jnp.float32),
            pltpu.VMEM((CK, QW), jnp.float32),
            pltpu.VMEM((CK, QW), jnp.float32),
            pltpu.VMEM_SHARED((NPAD, QW), jnp.float32),
            pltpu.SemaphoreType.DMA,
            pltpu.SemaphoreType.DMA,
            pltpu.SemaphoreType.DMA,
            pltpu.SemaphoreType.DMA,
        ],
    )
    out = call(table.reshape(NQ * 4 * N, QW), src3, attr3, dst3, zeros)
    return out.reshape(NQ, NPAD, QW)[:, :N]


def _prep_edges(edge_index, edge_attr):
    src = edge_index[0]
    dst = edge_index[1]
    a0 = edge_attr[:, 0]
    pad = EPAD - E
    src = jnp.concatenate([src, jnp.zeros((pad,), src.dtype)])
    a0 = jnp.concatenate([a0, jnp.zeros((pad,), a0.dtype)])
    dst = jnp.concatenate([dst, jnp.full((pad,), N, dst.dtype)])
    return (src.reshape(NT, ET), a0.reshape(NT, ET), dst.reshape(NT, ET))


def _mlp_body(agg_ref, s_ref, w1_ref, b1_ref, w2_ref, b2_ref, h_ref, st_ref):
    a = jnp.concatenate([agg_ref[q] for q in range(NQ)], axis=1) + s_ref[...]
    z = lax.dot_general(a, w1_ref[...], (((1,), (1,)), ((), ())),
                        preferred_element_type=jnp.float32) + b1_ref[...]
    z = jnp.maximum(z, 0.0)
    h = lax.dot_general(z, w2_ref[...], (((1,), (1,)), ((), ())),
                        preferred_element_type=jnp.float32) + b2_ref[...]
    h_ref[...] = h
    su = jnp.sum(h, axis=0, keepdims=True)
    sq = jnp.sum(h * h, axis=0, keepdims=True)
    part = jnp.concatenate([su, sq, jnp.zeros((6, D), jnp.float32)], axis=0)

    @pl.when(pl.program_id(0) == 0)
    def _():
        st_ref[...] = part

    @pl.when(pl.program_id(0) > 0)
    def _():
        st_ref[...] = st_ref[...] + part


def _mlp(agg, s, w1, b1, w2, b2):
    return pl.pallas_call(
        _mlp_body,
        grid=(NB,),
        in_specs=[
            pl.BlockSpec((NQ, BN, QW), lambda i: (0, i, 0)),
            pl.BlockSpec((BN, D), lambda i: (i, 0)),
            pl.BlockSpec((2 * D, D), lambda i: (0, 0)),
            pl.BlockSpec((1, 2 * D), lambda i: (0, 0)),
            pl.BlockSpec((D, 2 * D), lambda i: (0, 0)),
            pl.BlockSpec((1, D), lambda i: (0, 0)),
        ],
        out_specs=[
            pl.BlockSpec((BN, D), lambda i: (i, 0)),
            pl.BlockSpec((8, D), lambda i: (0, 0)),
        ],
        out_shape=[
            jax.ShapeDtypeStruct((N, D), jnp.float32),
            jax.ShapeDtypeStruct((8, D), jnp.float32),
        ],
    )(agg, s, w1, b1, w2, b2)


def _norm_body(h_ref, st_ref, g_ref, bt_ref, o_ref):
    mean = st_ref[0:1, :] * (1.0 / N)
    msq = st_ref[1:2, :] * (1.0 / N)
    var = msq - mean * mean
    inv = lax.rsqrt(var + EPS_BN)
    o_ref[...] = jnp.maximum(
        (h_ref[...] - mean) * inv * g_ref[...] + bt_ref[...], 0.0)


def _norm(h, st, gamma, beta):
    return pl.pallas_call(
        _norm_body,
        grid=(NB,),
        in_specs=[
            pl.BlockSpec((BN, D), lambda i: (i, 0)),
            pl.BlockSpec((8, D), lambda i: (0, 0)),
            pl.BlockSpec((1, D), lambda i: (0, 0)),
            pl.BlockSpec((1, D), lambda i: (0, 0)),
        ],
        out_specs=pl.BlockSpec((BN, D), lambda i: (i, 0)),
        out_shape=jax.ShapeDtypeStruct((N, D), jnp.float32),
    )(h, st, gamma, beta)


def _graph(x, edge_index, edge_attr, w_type, w1, b1, w2, b2, gamma, beta,
           zeros):
    table, s = _build_tables(x, w_type)
    src3, attr3, dst3 = _prep_edges(edge_index, edge_attr)
    agg = _sc_aggregate(table, src3, attr3, dst3, zeros)
    h, st = _mlp(agg, s, w1, b1.reshape(1, 2 * D), w2, b2.reshape(1, D))
    return _norm(h, st, gamma.reshape(1, D), beta.reshape(1, D))


def kernel(xA, edge_indexA, edge_attrA, xB, edge_indexB, edge_attrB,
           W_type, W1, b1, W2, b2, gamma, beta):
    zeros = jnp.zeros((ZR, QW), jnp.float32)
    outA = _graph(xA, edge_indexA, edge_attrA, W_type, W1, b1, W2, b2,
                  gamma, beta, zeros)
    outB = _graph(xB, edge_indexB, edge_attrB, W_type, W1, b1, W2, b2,
                  gamma, beta, zeros)
    return (outA, outB)
